# Initial kernel scaffold; baseline (speedup 1.0000x reference)
import functools

import jax
import jax.numpy as jnp
from jax import lax
from jax.experimental import pallas as pl
from jax.experimental.pallas import tpu as pltpu
from jax.experimental.pallas import tpu_sc as plsc

D = 32
B = 4096 * 200
NC, NS = 2, 16
NW = NC * NS
BPW = B // NW
GSIZE = 128
CHUNK = 1280
NG = CHUNK // GSIZE
NCHUNK = BPW // CHUNK
IDXROWS = BPW // GSIZE

_mesh = plsc.VectorSubcoreMesh(core_axis_name="c", subcore_axis_name="s")


@functools.partial(
    pl.kernel,
    out_type=jax.ShapeDtypeStruct((B, D), jnp.float32),
    mesh=_mesh,
    scratch_types=[
        pltpu.VMEM((IDXROWS, GSIZE), jnp.int32),
        pltpu.VMEM((CHUNK, D), jnp.float32),
        pltpu.SemaphoreType.DMA,
    ],
)
def _embed_gather(idx_hbm, table_hbm, out_hbm, idx_v, rows_v, sem):
    wid = lax.axis_index("s") * NC + lax.axis_index("c")
    pltpu.sync_copy(idx_hbm.at[pl.ds(wid * IDXROWS, IDXROWS)], idx_v)

    def chunk_body(g, carry):
        copies = []
        for j in range(NG):
            copies.append(
                pltpu.async_copy(
                    table_hbm.at[idx_v.at[g * NG + j]],
                    rows_v.at[pl.ds(j * GSIZE, GSIZE)],
                    sem,
                )
            )
        for cp in copies:
            cp.wait()
        pltpu.sync_copy(rows_v, out_hbm.at[pl.ds(wid * BPW + g * CHUNK, CHUNK)])
        return carry

    lax.fori_loop(0, NCHUNK, chunk_body, 0)


def kernel(input_ids, table):
    idx = input_ids.reshape(NW * IDXROWS, GSIZE)
    out = _embed_gather(idx, table)
    return out.reshape(input_ids.shape[0], input_ids.shape[1], D)


# speedup vs baseline: 1.4835x; 1.4835x over previous
import functools

import jax
import jax.numpy as jnp
from jax import lax
from jax.experimental import pallas as pl
from jax.experimental.pallas import tpu as pltpu
from jax.experimental.pallas import tpu_sc as plsc

D = 32
B = 4096 * 200
NC, NS = 2, 16
NW = NC * NS
BPW = B // NW
GSIZE = 128
CHUNK = 1280
NG = CHUNK // GSIZE
NCHUNK = BPW // CHUNK
IDXROWS = BPW // GSIZE

_mesh = plsc.VectorSubcoreMesh(core_axis_name="c", subcore_axis_name="s")


@functools.partial(
    pl.kernel,
    out_type=jax.ShapeDtypeStruct((B, D), jnp.float32),
    mesh=_mesh,
    scratch_types=[
        pltpu.VMEM((IDXROWS, GSIZE), jnp.int32),
        pltpu.VMEM((CHUNK, D), jnp.float32),
        pltpu.SemaphoreType.DMA,
    ],
    compiler_params=pltpu.CompilerParams(use_tc_tiling_on_sc=False),
)
def _embed_gather(idx_hbm, table_hbm, out_hbm, idx_v, rows_v, sem):
    wid = lax.axis_index("s") * NC + lax.axis_index("c")
    pltpu.sync_copy(idx_hbm.at[pl.ds(wid * IDXROWS, IDXROWS)], idx_v)

    def chunk_body(g, carry):
        copies = []
        for j in range(NG):
            copies.append(
                pltpu.async_copy(
                    table_hbm.at[idx_v.at[g * NG + j]],
                    rows_v.at[pl.ds(j * GSIZE, GSIZE)],
                    sem,
                )
            )
        for cp in copies:
            cp.wait()
        pltpu.sync_copy(rows_v, out_hbm.at[pl.ds(wid * BPW + g * CHUNK, CHUNK)])
        return carry

    lax.fori_loop(0, NCHUNK, chunk_body, 0)


def kernel(input_ids, table):
    idx = input_ids.reshape(NW * IDXROWS, GSIZE)
    out = _embed_gather(idx, table)
    return out.reshape(input_ids.shape[0], input_ids.shape[1], D)


# speedup vs baseline: 1.4932x; 1.0066x over previous
import functools

import jax
import jax.numpy as jnp
from jax import lax
from jax.experimental import pallas as pl
from jax.experimental.pallas import tpu as pltpu
from jax.experimental.pallas import tpu_sc as plsc

D = 32
B = 4096 * 200
NC, NS = 2, 16
NW = NC * NS
BPW = B // NW
GSIZE = 128
CHUNK = 640
NG = CHUNK // GSIZE
NCHUNK = BPW // CHUNK
NBUF = 4
IDXROWS = BPW // GSIZE

_mesh = plsc.VectorSubcoreMesh(core_axis_name="c", subcore_axis_name="s")


@functools.partial(
    pl.kernel,
    out_type=jax.ShapeDtypeStruct((B, D), jnp.float32),
    mesh=_mesh,
    scratch_types=(
        [pltpu.VMEM((IDXROWS, GSIZE), jnp.int32)]
        + [pltpu.VMEM((CHUNK, D), jnp.float32) for _ in range(NBUF)]
        + [pltpu.SemaphoreType.DMA for _ in range(2 * NBUF)]
    ),
    compiler_params=pltpu.CompilerParams(use_tc_tiling_on_sc=False),
)
def _embed_gather(idx_hbm, table_hbm, out_hbm, idx_v, *rest):
    bufs = rest[:NBUF]
    gsems = rest[NBUF:2 * NBUF]
    ssems = rest[2 * NBUF:]
    wid = lax.axis_index("s") * NC + lax.axis_index("c")
    base = wid * BPW
    pltpu.sync_copy(idx_hbm.at[pl.ds(wid * IDXROWS, IDXROWS)], idx_v)

    def fire_gathers(g, b):
        for j in range(NG):
            pltpu.async_copy(
                table_hbm.at[idx_v.at[g * NG + j]],
                bufs[b].at[pl.ds(j * GSIZE, GSIZE)],
                gsems[b],
            )

    def wait_gathers(b):
        pltpu.make_async_copy(table_hbm.at[pl.ds(0, CHUNK)], bufs[b], gsems[b]).wait()

    def fire_store(g, b):
        pltpu.async_copy(bufs[b], out_hbm.at[pl.ds(base + g * CHUNK, CHUNK)], ssems[b])

    def wait_store(b):
        pltpu.make_async_copy(bufs[b], out_hbm.at[pl.ds(0, CHUNK)], ssems[b]).wait()

    for b in range(NBUF):
        fire_gathers(b, b)

    @pl.loop(0, NCHUNK - NBUF, step=NBUF)
    def _pipeline(g0):
        for b in range(NBUF):
            wait_gathers(b)
            fire_store(g0 + b, b)
        for b in range(NBUF):
            wait_store(b)
            fire_gathers(g0 + b + NBUF, b)

    for b in range(NBUF):
        wait_gathers(b)
        fire_store(NCHUNK - NBUF + b, b)
    for b in range(NBUF):
        wait_store(b)


def kernel(input_ids, table):
    idx = input_ids.reshape(NW * IDXROWS, GSIZE)
    out = _embed_gather(idx, table)
    return out.reshape(input_ids.shape[0], input_ids.shape[1], D)
